# baseline (speedup 1.0000x reference)
import functools

import jax
import jax.numpy as jnp
from jax import lax
from jax.experimental import pallas as pl
from jax.experimental.pallas import tpu as pltpu
from jax.experimental.pallas import tpu_sc as plsc

_EMBED = 128
_SEQLEN = 200
_NSEG = 3
_NC = 2
_NS = 16
_NW = _NC * _NS
_LANES = 16
_CHUNK = 128


def _pe_seg_body(seg_ref, out_ref):
    rows = _SEQLEN * _NSEG
    r = lax.broadcasted_iota(jnp.int32, (rows, _EMBED), 0)
    c = lax.broadcasted_iota(jnp.int32, (rows, _EMBED), 1)
    pos = (r // _NSEG).astype(jnp.float32)
    g = r % _NSEG
    i_even = ((c // 2) * 2).astype(jnp.float32)
    div = jnp.exp(-jnp.log(10000.0) * i_even / _EMBED)
    ang = pos * div
    pe = jnp.where(c % 2 == 0, jnp.sin(ang), jnp.cos(ang))
    seg = jnp.where(
        g == 0,
        seg_ref[0:1, :],
        jnp.where(g == 1, seg_ref[1:2, :], seg_ref[2:3, :]),
    )
    out_ref[...] = pe + seg


def _build_pe_seg(segment_table):
    return pl.pallas_call(
        _pe_seg_body,
        out_shape=jax.ShapeDtypeStruct((_SEQLEN * _NSEG, _EMBED), jnp.float32),
    )(segment_table)


def _make_sc_kernel(n_rows):
    rows_per_w = n_rows // _NW
    n_chunks = rows_per_w // _CHUNK
    mesh = plsc.VectorSubcoreMesh(core_axis_name="c", subcore_axis_name="s")

    @functools.partial(
        pl.kernel,
        out_type=jax.ShapeDtypeStruct((n_rows, _EMBED), jnp.float32),
        mesh=mesh,
        scratch_types=[
            pltpu.VMEM((_CHUNK,), jnp.int32),
            pltpu.VMEM((_CHUNK,), jnp.int32),
            pltpu.VMEM((_CHUNK, _EMBED), jnp.float32),
            pltpu.VMEM((_SEQLEN * _NSEG, _EMBED), jnp.float32),
            pltpu.SemaphoreType.DMA,
        ],
    )
    def sc_embed(seq_hbm, lbl_hbm, tok_hbm, peseg_hbm, out_hbm,
                 idx_tok, idx_ps, rows_tok, peseg_l, sem_a):
        wid = lax.axis_index("s") * _NC + lax.axis_index("c")
        wbase = wid * rows_per_w
        pltpu.sync_copy(peseg_hbm, peseg_l)

        @pl.loop(0, n_chunks)
        def _chunk(ci):
            base = wbase + ci * _CHUNK
            pltpu.sync_copy(seq_hbm.at[pl.ds(base, _CHUNK)], idx_tok)
            pltpu.sync_copy(lbl_hbm.at[pl.ds(base, _CHUNK)], idx_ps)

            @pl.loop(0, _CHUNK // _LANES)
            def _fuse(j):
                off = base + j * _LANES
                pos = (off + lax.iota(jnp.int32, _LANES)) % _SEQLEN
                sl = pl.ds(j * _LANES, _LANES)
                idx_ps[sl] = pos * _NSEG + idx_ps[sl]

            pltpu.async_copy(tok_hbm.at[idx_tok], rows_tok, sem_a).wait()

            @pl.loop(0, _CHUNK // _LANES)
            def _add(g):
                qv = idx_ps[pl.ds(g * _LANES, _LANES)]
                for r in range(_LANES):
                    q = qv[r]
                    row = g * _LANES + r
                    for cg in range(_EMBED // _LANES):
                        sl = pl.ds(cg * _LANES, _LANES)
                        plsc.addupdate(rows_tok.at[row, sl], peseg_l[q, sl])

            pltpu.sync_copy(rows_tok, out_hbm.at[pl.ds(base, _CHUNK)])

    return sc_embed


def kernel(sequence, segment_label, token_table, segment_table):
    batch, seqlen = sequence.shape
    n_rows = batch * seqlen
    pe_seg = _build_pe_seg(segment_table.astype(jnp.float32))
    seq_flat = sequence.reshape(-1).astype(jnp.int32)
    lbl_flat = segment_label.reshape(-1).astype(jnp.int32)
    sc = _make_sc_kernel(n_rows)
    out = sc(seq_flat, lbl_flat, token_table, pe_seg)
    return out.reshape(batch, seqlen, _EMBED)


# speedup vs baseline: 1.7175x; 1.7175x over previous
import functools

import jax
import jax.numpy as jnp
from jax import lax
from jax.experimental import pallas as pl
from jax.experimental.pallas import tpu as pltpu
from jax.experimental.pallas import tpu_sc as plsc

_EMBED = 128
_SEQLEN = 200
_NSEG = 3
_NC = 2
_NS = 16
_NW = _NC * _NS
_LANES = 16
_CHUNK = 128


def _pe_seg_body(seg_ref, out_ref):
    rows = _SEQLEN * _NSEG
    r = lax.broadcasted_iota(jnp.int32, (rows, _EMBED), 0)
    c = lax.broadcasted_iota(jnp.int32, (rows, _EMBED), 1)
    pos = (r // _NSEG).astype(jnp.float32)
    g = r % _NSEG
    i_even = ((c // 2) * 2).astype(jnp.float32)
    div = jnp.exp(-jnp.log(10000.0) * i_even / _EMBED)
    ang = pos * div
    pe = jnp.where(c % 2 == 0, jnp.sin(ang), jnp.cos(ang))
    seg = jnp.where(
        g == 0,
        seg_ref[0:1, :],
        jnp.where(g == 1, seg_ref[1:2, :], seg_ref[2:3, :]),
    )
    out_ref[...] = pe + seg


def _build_pe_seg(segment_table):
    return pl.pallas_call(
        _pe_seg_body,
        out_shape=jax.ShapeDtypeStruct((_SEQLEN * _NSEG, _EMBED), jnp.float32),
    )(segment_table)


def _make_sc_kernel(n_rows):
    rows_per_w = n_rows // _NW
    n_chunks = rows_per_w // _CHUNK
    mesh = plsc.VectorSubcoreMesh(core_axis_name="c", subcore_axis_name="s")

    @functools.partial(
        pl.kernel,
        out_type=jax.ShapeDtypeStruct((n_rows, _EMBED), jnp.float32),
        mesh=mesh,
        scratch_types=[
            pltpu.VMEM((_CHUNK,), jnp.int32),
            pltpu.VMEM((_CHUNK,), jnp.int32),
            pltpu.VMEM((_CHUNK, _EMBED), jnp.float32),
            pltpu.VMEM_SHARED((_SEQLEN * _NSEG, _EMBED), jnp.float32),
            pltpu.SemaphoreType.DMA,
        ],
    )
    def sc_embed(seq_hbm, lbl_hbm, tok_hbm, peseg_hbm, out_hbm,
                 idx_tok, idx_ps, rows_tok, peseg_l, sem_a):
        sid = lax.axis_index("s")
        wid = sid * _NC + lax.axis_index("c")
        wbase = wid * rows_per_w

        @pl.when(sid == 0)
        def _stage():
            pltpu.sync_copy(peseg_hbm, peseg_l)

        plsc.subcore_barrier()

        @pl.loop(0, n_chunks)
        def _chunk(ci):
            base = wbase + ci * _CHUNK
            pltpu.sync_copy(seq_hbm.at[pl.ds(base, _CHUNK)], idx_tok)
            pltpu.sync_copy(lbl_hbm.at[pl.ds(base, _CHUNK)], idx_ps)

            @pl.loop(0, _CHUNK // _LANES)
            def _fuse(j):
                off = base + j * _LANES
                pos = (off + lax.iota(jnp.int32, _LANES)) % _SEQLEN
                sl = pl.ds(j * _LANES, _LANES)
                idx_ps[sl] = pos * _NSEG + idx_ps[sl]

            pltpu.async_copy(peseg_l.at[idx_ps], rows_tok, sem_a).wait()
            pltpu.async_copy(tok_hbm.at[idx_tok], rows_tok, sem_a, add=True).wait()

            pltpu.sync_copy(rows_tok, out_hbm.at[pl.ds(base, _CHUNK)])

    return sc_embed


def kernel(sequence, segment_label, token_table, segment_table):
    batch, seqlen = sequence.shape
    n_rows = batch * seqlen
    pe_seg = _build_pe_seg(segment_table.astype(jnp.float32))
    seq_flat = sequence.reshape(-1).astype(jnp.int32)
    lbl_flat = segment_label.reshape(-1).astype(jnp.int32)
    sc = _make_sc_kernel(n_rows)
    out = sc(seq_flat, lbl_flat, token_table, pe_seg)
    return out.reshape(batch, seqlen, _EMBED)


# speedup vs baseline: 2.3655x; 1.3773x over previous
import functools

import jax
import jax.numpy as jnp
from jax import lax
from jax.experimental import pallas as pl
from jax.experimental.pallas import tpu as pltpu
from jax.experimental.pallas import tpu_sc as plsc

_EMBED = 128
_SEQLEN = 200
_NSEG = 3
_NC = 2
_NS = 16
_NW = _NC * _NS
_LANES = 16
_CHUNK = 128


def _pe_seg_body(seg_ref, out_ref):
    rows = _SEQLEN * _NSEG
    r = lax.broadcasted_iota(jnp.int32, (rows, _EMBED), 0)
    c = lax.broadcasted_iota(jnp.int32, (rows, _EMBED), 1)
    pos = (r // _NSEG).astype(jnp.float32)
    g = r % _NSEG
    i_even = ((c // 2) * 2).astype(jnp.float32)
    div = jnp.exp(-jnp.log(10000.0) * i_even / _EMBED)
    ang = pos * div
    pe = jnp.where(c % 2 == 0, jnp.sin(ang), jnp.cos(ang))
    seg = jnp.where(
        g == 0,
        seg_ref[0:1, :],
        jnp.where(g == 1, seg_ref[1:2, :], seg_ref[2:3, :]),
    )
    out_ref[...] = pe + seg


def _build_pe_seg(segment_table):
    return pl.pallas_call(
        _pe_seg_body,
        out_shape=jax.ShapeDtypeStruct((_SEQLEN * _NSEG, _EMBED), jnp.float32),
    )(segment_table)


def _make_sc_kernel(n_rows):
    rows_per_w = n_rows // _NW
    n_chunks = rows_per_w // _CHUNK
    n_pairs = n_chunks // 2
    mesh = plsc.VectorSubcoreMesh(core_axis_name="c", subcore_axis_name="s")

    @functools.partial(
        pl.kernel,
        out_type=jax.ShapeDtypeStruct((n_rows, _EMBED), jnp.float32),
        mesh=mesh,
        scratch_types=[
            pltpu.VMEM((_CHUNK,), jnp.int32),
            pltpu.VMEM((_CHUNK,), jnp.int32),
            pltpu.VMEM((_CHUNK,), jnp.int32),
            pltpu.VMEM((_CHUNK,), jnp.int32),
            pltpu.VMEM((_CHUNK, _EMBED), jnp.float32),
            pltpu.VMEM((_CHUNK, _EMBED), jnp.float32),
            pltpu.VMEM_SHARED((_SEQLEN * _NSEG, _EMBED), jnp.float32),
            pltpu.SemaphoreType.DMA,
            pltpu.SemaphoreType.DMA,
            pltpu.SemaphoreType.DMA,
            pltpu.SemaphoreType.DMA,
            pltpu.SemaphoreType.DMA,
            pltpu.SemaphoreType.DMA,
        ],
    )
    def sc_embed(seq_hbm, lbl_hbm, tok_hbm, peseg_hbm, out_hbm,
                 itok0, itok1, ips0, ips1, rows0, rows1, peseg_l,
                 sem_ps0, sem_ps1, sem_tok0, sem_tok1, sem_out0, sem_out1):
        sid = lax.axis_index("s")
        wid = sid * _NC + lax.axis_index("c")
        wbase = wid * rows_per_w

        @pl.when(sid == 0)
        def _stage():
            pltpu.sync_copy(peseg_hbm, peseg_l)

        plsc.subcore_barrier()

        def fetch_fuse(ci, itok, ips):
            base = wbase + ci * _CHUNK
            pltpu.sync_copy(seq_hbm.at[pl.ds(base, _CHUNK)], itok)
            pltpu.sync_copy(lbl_hbm.at[pl.ds(base, _CHUNK)], ips)

            @pl.loop(0, _CHUNK // _LANES)
            def _fuse(j):
                off = base + j * _LANES
                pos = (off + lax.iota(jnp.int32, _LANES)) % _SEQLEN
                sl = pl.ds(j * _LANES, _LANES)
                ips[sl] = pos * _NSEG + ips[sl]

        def start_ps(ips, rows, sem):
            pltpu.async_copy(peseg_l.at[ips], rows, sem)

        def wait_ps(ips, rows, sem):
            pltpu.make_async_copy(peseg_l.at[ips], rows, sem).wait()

        def start_tok(itok, rows, sem):
            pltpu.async_copy(tok_hbm.at[itok], rows, sem, add=True)

        def wait_tok(itok, rows, sem):
            pltpu.make_async_copy(tok_hbm.at[itok], rows, sem).wait()

        def start_out(ci, rows, sem):
            base = wbase + ci * _CHUNK
            pltpu.async_copy(rows, out_hbm.at[pl.ds(base, _CHUNK)], sem)

        def wait_out(rows, sem):
            pltpu.make_async_copy(
                rows, out_hbm.at[pl.ds(wbase, _CHUNK)], sem).wait()

        fetch_fuse(0, itok0, ips0)
        start_ps(ips0, rows0, sem_ps0)

        @pl.loop(0, n_pairs)
        def _pair(h):
            e = 2 * h
            o = 2 * h + 1

            fetch_fuse(o, itok1, ips1)

            wait_ps(ips0, rows0, sem_ps0)
            start_tok(itok0, rows0, sem_tok0)

            @pl.when(h > 0)
            def _w_prev_odd():
                wait_out(rows1, sem_out1)

            start_ps(ips1, rows1, sem_ps1)

            wait_tok(itok0, rows0, sem_tok0)
            start_out(e, rows0, sem_out0)

            @pl.when(h + 1 < n_pairs)
            def _prefetch_even():
                fetch_fuse(e + 2, itok0, ips0)

            wait_ps(ips1, rows1, sem_ps1)
            start_tok(itok1, rows1, sem_tok1)

            wait_out(rows0, sem_out0)

            @pl.when(h + 1 < n_pairs)
            def _next_even_ps():
                start_ps(ips0, rows0, sem_ps0)

            wait_tok(itok1, rows1, sem_tok1)
            start_out(o, rows1, sem_out1)

        wait_out(rows1, sem_out1)

    return sc_embed


def kernel(sequence, segment_label, token_table, segment_table):
    batch, seqlen = sequence.shape
    n_rows = batch * seqlen
    pe_seg = _build_pe_seg(segment_table.astype(jnp.float32))
    seq_flat = sequence.reshape(-1).astype(jnp.int32)
    lbl_flat = segment_label.reshape(-1).astype(jnp.int32)
    sc = _make_sc_kernel(n_rows)
    out = sc(seq_flat, lbl_flat, token_table, pe_seg)
    return out.reshape(batch, seqlen, _EMBED)


# speedup vs baseline: 3.4018x; 1.4381x over previous
import functools

import jax
import jax.numpy as jnp
from jax import lax
from jax.experimental import pallas as pl
from jax.experimental.pallas import tpu as pltpu
from jax.experimental.pallas import tpu_sc as plsc

_EMBED = 128
_SEQLEN = 200
_NSEG = 3
_NC = 2
_NS = 16
_NW = _NC * _NS
_LANES = 16
_CHUNK = 128


def _pe_seg_body(seg_ref, out_ref):
    rows = _SEQLEN * _NSEG
    r = lax.broadcasted_iota(jnp.int32, (rows, _EMBED), 0)
    c = lax.broadcasted_iota(jnp.int32, (rows, _EMBED), 1)
    pos = (r // _NSEG).astype(jnp.float32)
    g = r % _NSEG
    i_even = ((c // 2) * 2).astype(jnp.float32)
    div = jnp.exp(-jnp.log(10000.0) * i_even / _EMBED)
    ang = pos * div
    pe = jnp.where(c % 2 == 0, jnp.sin(ang), jnp.cos(ang))
    seg = jnp.where(
        g == 0,
        seg_ref[0:1, :],
        jnp.where(g == 1, seg_ref[1:2, :], seg_ref[2:3, :]),
    )
    out_ref[...] = pe + seg


def _build_pe_seg(segment_table):
    return pl.pallas_call(
        _pe_seg_body,
        out_shape=jax.ShapeDtypeStruct((_SEQLEN * _NSEG, _EMBED), jnp.float32),
    )(segment_table)


def _make_sc_kernel(n_rows):
    rows_per_w = n_rows // _NW
    n_chunks = rows_per_w // _CHUNK
    n_pairs = n_chunks // 2
    mesh = plsc.VectorSubcoreMesh(core_axis_name="c", subcore_axis_name="s")

    @functools.partial(
        pl.kernel,
        out_type=jax.ShapeDtypeStruct((n_rows, _EMBED), jnp.float32),
        mesh=mesh,
        scratch_types=[
            pltpu.VMEM((n_chunks, _CHUNK), jnp.int32),
            pltpu.VMEM((n_chunks, _CHUNK), jnp.int32),
            pltpu.VMEM((_CHUNK, _EMBED), jnp.float32),
            pltpu.VMEM((_CHUNK, _EMBED), jnp.float32),
            pltpu.VMEM_SHARED((_SEQLEN * _NSEG, _EMBED), jnp.float32),
            pltpu.SemaphoreType.DMA,
            pltpu.SemaphoreType.DMA,
            pltpu.SemaphoreType.DMA,
            pltpu.SemaphoreType.DMA,
            pltpu.SemaphoreType.DMA,
            pltpu.SemaphoreType.DMA,
        ],
    )
    def sc_embed(seq_hbm, lbl_hbm, tok_hbm, peseg_hbm, out_hbm,
                 itok_a, ips_a, rows0, rows1, peseg_l,
                 sem_ps0, sem_ps1, sem_tok0, sem_tok1, sem_out0, sem_out1):
        sid = lax.axis_index("s")
        wid = sid * _NC + lax.axis_index("c")
        wbase = wid * rows_per_w

        @pl.when(sid == 0)
        def _stage():
            pltpu.sync_copy(peseg_hbm, peseg_l)

        pltpu.sync_copy(seq_hbm.at[wid], itok_a)
        pltpu.sync_copy(lbl_hbm.at[wid], ips_a)

        @pl.loop(0, n_chunks)
        def _fuse_chunk(ci):
            @pl.loop(0, _CHUNK // _LANES)
            def _fuse(j):
                off = wbase + ci * _CHUNK + j * _LANES
                pos = (off + lax.iota(jnp.int32, _LANES)) % _SEQLEN
                sl = pl.ds(j * _LANES, _LANES)
                ips_a[ci, sl] = pos * _NSEG + ips_a[ci, sl]

        plsc.subcore_barrier()

        def start_ps(ci, rows, sem):
            pltpu.async_copy(peseg_l.at[ips_a.at[ci]], rows, sem)

        def wait_ps(rows, sem):
            pltpu.make_async_copy(peseg_l.at[ips_a.at[0]], rows, sem).wait()

        def start_tok(ci, rows, sem):
            pltpu.async_copy(tok_hbm.at[itok_a.at[ci]], rows, sem, add=True)

        def wait_tok(rows, sem):
            pltpu.make_async_copy(tok_hbm.at[itok_a.at[0]], rows, sem).wait()

        def start_out(ci, rows, sem):
            base = wbase + ci * _CHUNK
            pltpu.async_copy(rows, out_hbm.at[pl.ds(base, _CHUNK)], sem)

        def wait_out(rows, sem):
            pltpu.make_async_copy(
                rows, out_hbm.at[pl.ds(wbase, _CHUNK)], sem).wait()

        start_ps(0, rows0, sem_ps0)

        @pl.loop(0, n_pairs)
        def _pair(h):
            e = 2 * h
            o = 2 * h + 1

            wait_ps(rows0, sem_ps0)
            start_tok(e, rows0, sem_tok0)

            @pl.when(h > 0)
            def _w_prev_odd():
                wait_out(rows1, sem_out1)

            start_ps(o, rows1, sem_ps1)

            wait_tok(rows0, sem_tok0)
            start_out(e, rows0, sem_out0)

            wait_ps(rows1, sem_ps1)
            start_tok(o, rows1, sem_tok1)

            wait_out(rows0, sem_out0)

            @pl.when(h + 1 < n_pairs)
            def _next_even_ps():
                start_ps(e + 2, rows0, sem_ps0)

            wait_tok(rows1, sem_tok1)
            start_out(o, rows1, sem_out1)

        wait_out(rows1, sem_out1)

    return sc_embed


def kernel(sequence, segment_label, token_table, segment_table):
    batch, seqlen = sequence.shape
    n_rows = batch * seqlen
    pe_seg = _build_pe_seg(segment_table.astype(jnp.float32))
    rows_per_w = n_rows // _NW
    n_chunks = rows_per_w // _CHUNK
    seq_w = sequence.reshape(_NW, n_chunks, _CHUNK).astype(jnp.int32)
    lbl_w = segment_label.reshape(_NW, n_chunks, _CHUNK).astype(jnp.int32)
    sc = _make_sc_kernel(n_rows)
    out = sc(seq_w, lbl_w, token_table, pe_seg)
    return out.reshape(batch, seqlen, _EMBED)


# speedup vs baseline: 3.8983x; 1.1459x over previous
import functools

import jax
import jax.numpy as jnp
from jax import lax
from jax.experimental import pallas as pl
from jax.experimental.pallas import tpu as pltpu
from jax.experimental.pallas import tpu_sc as plsc

_EMBED = 128
_SEQLEN = 200
_NSEG = 3
_NC = 2
_NS = 16
_NW = _NC * _NS
_LANES = 16
_CHUNK = 128


def _pe_seg_body(seg_ref, out_ref):
    rows = _SEQLEN * _NSEG
    r = lax.broadcasted_iota(jnp.int32, (rows, _EMBED), 0)
    c = lax.broadcasted_iota(jnp.int32, (rows, _EMBED), 1)
    pos = (r // _NSEG).astype(jnp.float32)
    g = r % _NSEG
    i_even = ((c // 2) * 2).astype(jnp.float32)
    div = jnp.exp(-jnp.log(10000.0) * i_even / _EMBED)
    ang = pos * div
    pe = jnp.where(c % 2 == 0, jnp.sin(ang), jnp.cos(ang))
    seg = jnp.where(
        g == 0,
        seg_ref[0:1, :],
        jnp.where(g == 1, seg_ref[1:2, :], seg_ref[2:3, :]),
    )
    out_ref[...] = pe + seg


def _build_pe_seg(segment_table):
    return pl.pallas_call(
        _pe_seg_body,
        out_shape=jax.ShapeDtypeStruct((_SEQLEN * _NSEG, _EMBED), jnp.float32),
    )(segment_table)


def _make_sc_kernel(n_rows):
    rows_per_w = n_rows // _NW
    n_chunks = rows_per_w // _CHUNK
    n_pairs = n_chunks // 2
    mesh = plsc.VectorSubcoreMesh(core_axis_name="c", subcore_axis_name="s")

    @functools.partial(
        pl.kernel,
        out_type=jax.ShapeDtypeStruct((n_rows, _EMBED), jnp.float32),
        mesh=mesh,
        scratch_types=[
            pltpu.VMEM((n_chunks, _CHUNK), jnp.int32),
            pltpu.VMEM((n_chunks, _CHUNK), jnp.int32),
            pltpu.VMEM((_CHUNK, _EMBED), jnp.float32),
            pltpu.VMEM((_CHUNK, _EMBED), jnp.float32),
            pltpu.VMEM((_CHUNK, _EMBED), jnp.float32),
            pltpu.VMEM((_CHUNK, _EMBED), jnp.float32),
            pltpu.VMEM_SHARED((_SEQLEN * _NSEG, _EMBED), jnp.float32),
            pltpu.SemaphoreType.DMA,
            pltpu.SemaphoreType.DMA,
            pltpu.SemaphoreType.DMA,
            pltpu.SemaphoreType.DMA,
            pltpu.SemaphoreType.DMA,
            pltpu.SemaphoreType.DMA,
            pltpu.SemaphoreType.DMA,
            pltpu.SemaphoreType.DMA,
            pltpu.SemaphoreType.DMA,
            pltpu.SemaphoreType.DMA,
            pltpu.SemaphoreType.DMA,
            pltpu.SemaphoreType.DMA,
        ],
    )
    def sc_embed(seq_hbm, lbl_hbm, tok_hbm, peseg_hbm, out_hbm,
                 itok_a, ips_a, rows0, rows1, rows2, rows3, peseg_l,
                 sem_ps0, sem_ps1, sem_ps2, sem_ps3,
                 sem_tok0, sem_tok1, sem_tok2, sem_tok3,
                 sem_out0, sem_out1, sem_out2, sem_out3):
        sid = lax.axis_index("s")
        wid = sid * _NC + lax.axis_index("c")
        wbase = wid * rows_per_w

        @pl.when(sid == 0)
        def _stage():
            pltpu.sync_copy(peseg_hbm, peseg_l)

        pltpu.sync_copy(seq_hbm.at[wid], itok_a)
        pltpu.sync_copy(lbl_hbm.at[wid], ips_a)

        @pl.loop(0, n_chunks)
        def _fuse_chunk(ci):
            @pl.loop(0, _CHUNK // _LANES)
            def _fuse(j):
                off = wbase + ci * _CHUNK + j * _LANES
                pos = (off + lax.iota(jnp.int32, _LANES)) % _SEQLEN
                sl = pl.ds(j * _LANES, _LANES)
                ips_a[ci, sl] = pos * _NSEG + ips_a[ci, sl]

        plsc.subcore_barrier()

        rows = [rows0, rows1, rows2, rows3]
        sem_ps = [sem_ps0, sem_ps1, sem_ps2, sem_ps3]
        sem_tok = [sem_tok0, sem_tok1, sem_tok2, sem_tok3]
        sem_out = [sem_out0, sem_out1, sem_out2, sem_out3]

        def start_ps(ci, b):
            pltpu.async_copy(peseg_l.at[ips_a.at[ci]], rows[b], sem_ps[b])

        def wait_ps(b):
            pltpu.make_async_copy(
                peseg_l.at[ips_a.at[0]], rows[b], sem_ps[b]).wait()

        def start_tok(ci, b):
            pltpu.async_copy(
                tok_hbm.at[itok_a.at[ci]], rows[b], sem_tok[b], add=True)

        def wait_tok(b):
            pltpu.make_async_copy(
                tok_hbm.at[itok_a.at[0]], rows[b], sem_tok[b]).wait()

        def start_out(ci, b):
            base = wbase + ci * _CHUNK
            pltpu.async_copy(rows[b], out_hbm.at[pl.ds(base, _CHUNK)],
                             sem_out[b])

        def wait_out(b):
            pltpu.make_async_copy(
                rows[b], out_hbm.at[pl.ds(wbase, _CHUNK)], sem_out[b]).wait()

        start_ps(0, 0)

        wait_ps(0)
        start_tok(0, 0)
        start_ps(1, 1)

        wait_ps(1)
        start_tok(1, 1)
        wait_tok(0)
        start_out(0, 0)
        start_ps(2, 2)

        wait_ps(2)
        start_tok(2, 2)
        wait_tok(1)
        start_out(1, 1)
        start_ps(3, 3)

        wait_ps(3)
        start_tok(3, 3)
        wait_tok(2)
        start_out(2, 2)
        wait_out(0)
        start_ps(4, 0)

        @pl.loop(1, n_chunks // 4)
        def _quad(t):
            for j in range(4):
                ci = 4 * t + j
                b = j
                wait_ps(b)
                start_tok(ci, b)
                wait_tok((b - 1) % 4)
                start_out(ci - 1, (b - 1) % 4)
                wait_out((b + 1) % 4)

                @pl.when(ci + 1 < n_chunks)
                def _refill():
                    start_ps(ci + 1, (b + 1) % 4)

        wait_tok(3)
        start_out(n_chunks - 1, 3)
        wait_out(1)
        wait_out(2)
        wait_out(3)

    return sc_embed


def kernel(sequence, segment_label, token_table, segment_table):
    batch, seqlen = sequence.shape
    n_rows = batch * seqlen
    pe_seg = _build_pe_seg(segment_table.astype(jnp.float32))
    rows_per_w = n_rows // _NW
    n_chunks = rows_per_w // _CHUNK
    seq_w = sequence.reshape(_NW, n_chunks, _CHUNK).astype(jnp.int32)
    lbl_w = segment_label.reshape(_NW, n_chunks, _CHUNK).astype(jnp.int32)
    sc = _make_sc_kernel(n_rows)
    out = sc(seq_w, lbl_w, token_table, pe_seg)
    return out.reshape(batch, seqlen, _EMBED)
